# baseline (speedup 1.0000x reference)
import jax
import jax.numpy as jnp
from jax import lax
from jax.experimental import pallas as pl
from jax.experimental.pallas import tpu as pltpu

N, T, K, L = 32, 512, 4096, 64
SOFT = 1e-10


def _body(probs_ref, tgt_ref, out_ref):
    n = pl.program_id(0)
    x = probs_ref[0]
    s = jnp.sum(x, axis=0, keepdims=True) + T * SOFT
    logm = jnp.log(s / T)
    tgt = tgt_ref[0]
    k_iota = lax.broadcasted_iota(jnp.int32, (L, K), 1)
    onehot = k_iota == jnp.broadcast_to(tgt, (L, K))
    contrib = jnp.sum(jnp.where(onehot, jnp.broadcast_to(logm, (L, K)), 0.0))

    @pl.when(n == 0)
    def _():
        out_ref[...] = jnp.zeros_like(out_ref)

    out_ref[...] += (-contrib / (N * T)).reshape(1, 1)


def kernel(probs, targets):
    tgt3 = targets.astype(jnp.int32).reshape(N, L, 1)
    out = pl.pallas_call(
        _body,
        grid=(N,),
        in_specs=[
            pl.BlockSpec((1, T, K), lambda n: (n, 0, 0)),
            pl.BlockSpec((1, L, 1), lambda n: (n, 0, 0)),
        ],
        out_specs=pl.BlockSpec((1, 1), lambda n: (0, 0)),
        out_shape=jax.ShapeDtypeStruct((1, 1), jnp.float32),
    )(probs, tgt3)
    return out[0, 0]
